# baseline (speedup 1.0000x reference)
import functools

import jax
import jax.numpy as jnp
from jax import lax
from jax.experimental import pallas as pl
from jax.experimental.pallas import tpu as pltpu
from jax.experimental.pallas import tpu_sc as plsc

_KS2INV = 1.0 / (0.2 * 0.2)


def _b16(a):
    return a.astype(jnp.bfloat16)


def _dot16(a, b):
    return lax.dot_general(_b16(a), _b16(b), (((1,), (0,)), ((), ())),
                           preferred_element_type=jnp.float32)
_NC = 2
_NS = 16
_NW = _NC * _NS
_TW = 128


def _cdiv(a, b):
    return -(-a // b)


def _chunking(bpw):
    ch = 128
    nc = _cdiv(bpw, ch)
    return nc, ch


def _sc_gather(table, idx2d, np_rows):
    k = idx2d.shape[1]
    bpw = np_rows * k // _NW
    nc, ch = _chunking(bpw)
    b_pad = _NW * nc * ch

    flat = jnp.pad(idx2d.astype(jnp.int32), ((0, np_rows - idx2d.shape[0]), (0, 0))).reshape(-1)
    flat = jnp.pad(flat, (0, b_pad - flat.shape[0]))

    mesh = plsc.VectorSubcoreMesh(core_axis_name="c", subcore_axis_name="s")

    @functools.partial(
        pl.kernel, mesh=mesh,
        out_type=jax.ShapeDtypeStruct((b_pad, _TW), jnp.float32),
        scratch_types=[
            pltpu.VMEM((ch,), jnp.int32),
            pltpu.VMEM((ch, _TW), jnp.float32),
            pltpu.SemaphoreType.DMA,
        ],
    )
    def gather_k(idx_hbm, table_hbm, out_hbm, idx_v, rows_v, sem):
        wid = lax.axis_index("s") * _NC + lax.axis_index("c")
        base = wid * (nc * ch)

        def body(c, carry):
            off = base + c * ch
            pltpu.sync_copy(idx_hbm.at[pl.ds(off, ch)], idx_v)
            pltpu.async_copy(table_hbm.at[idx_v], rows_v, sem).wait()
            pltpu.sync_copy(rows_v, out_hbm.at[pl.ds(off, ch)])
            return carry

        lax.fori_loop(0, nc, body, 0)

    return gather_k(flat, table)


def _gib_tc(rows, dst, obs, W, b, c_in, feat_off, nb, k, s_mode="2x8"):
    np_rows = dst.shape[0]
    c_out = W.shape[1]
    grid = np_rows // nb

    def body(rows_ref, dst_ref, obs_ref, w_ref, b_ref, out_ref):
        r3 = rows_ref[...].reshape(nb, k, _TW)
        nbc = r3[:, :, 0:3]
        nbf = r3[:, :, feat_off:feat_off + c_in]
        dstb = dst_ref[...]
        rel = (nbc - dstb[:, None, 0:3]).reshape(nb * k, 3)
        proj = lax.dot_general(_b16(rel), _b16(obs_ref[...]), (((1,), (1,)), ((), ())),
                               preferred_element_type=jnp.float32)
        logit = (-_KS2INV) * proj * proj
        l3 = logit.reshape(nb, k, 8)
        m = jnp.max(l3, axis=1, keepdims=True)
        e = jnp.exp(l3 - m)
        if s_mode == "2x8":
            h1 = e[:, 0, :]
            for kk in range(1, 8):
                h1 = h1 + e[:, kk, :]
            h2 = e[:, 8, :]
            for kk in range(9, 16):
                h2 = h2 + e[:, kk, :]
            s = (h1 + h2)[:, None, :]
        else:
            h1 = e[:, 0, :]
            for kk in range(1, 16):
                h1 = h1 + e[:, kk, :]
            s = h1[:, None, :]
        wgt = _b16(e / s).astype(jnp.float32)
        nbf16 = _b16(nbf).astype(jnp.float32)
        parts = [jnp.sum(wgt[:, :, o:o + 1] * nbf16, axis=1) for o in range(8)]
        agg = jnp.concatenate(parts, axis=-1)
        outf = _dot16(agg, w_ref[...]) + b_ref[...]
        outf = jnp.maximum(outf, 0.0)
        pad = jnp.zeros((nb, _TW - 16 - c_out), jnp.float32)
        out_ref[...] = jnp.concatenate([dstb[:, 0:16], outf, pad], axis=-1)

    return pl.pallas_call(
        body,
        grid=(grid,),
        in_specs=[
            pl.BlockSpec((nb * k, _TW), lambda i: (i, 0)),
            pl.BlockSpec((nb, _TW), lambda i: (i, 0)),
            pl.BlockSpec((8, 3), lambda i: (0, 0)),
            pl.BlockSpec(W.shape, lambda i: (0, 0)),
            pl.BlockSpec((1, c_out), lambda i: (0, 0)),
        ],
        out_specs=pl.BlockSpec((nb, _TW), lambda i: (i, 0)),
        out_shape=jax.ShapeDtypeStruct((np_rows, _TW), jnp.float32),
    )(rows, dst, obs, W, b.reshape(1, c_out))


def _dec_tc(rows, skip, c_in, c_skip, Wd, bd, nb):
    np_rows = skip.shape[0]
    c_out = Wd.shape[1]
    grid = np_rows // nb

    def body(rows_ref, skip_ref, wd_ref, bd_ref, out_ref):
        r3 = rows_ref[...].reshape(nb, 3, _TW)
        nbc = r3[:, :, 0:3]
        nbf = r3[:, :, 16:16 + c_in]
        skipb = skip_ref[...]
        rel = nbc - skipb[:, None, 0:3]
        sq = rel * rel
        dist = jnp.sqrt((sq[:, :, 0] + sq[:, :, 1]) + sq[:, :, 2] + 1e-12)
        w = 1.0 / (dist + 1e-8)
        ws = (w[:, 0] + w[:, 1]) + w[:, 2]
        w = w / ws[:, None]
        pu = w[:, :, None] * nbf
        upf = (pu[:, 0, :] + pu[:, 1, :]) + pu[:, 2, :]
        cat = jnp.concatenate([upf, skipb[:, 16:16 + c_skip]], axis=-1)
        h = _dot16(cat, wd_ref[...]) + bd_ref[...]
        h = jnp.maximum(h, 0.0)
        pad = jnp.zeros((nb, _TW - 16 - c_out), jnp.float32)
        out_ref[...] = jnp.concatenate([skipb[:, 0:16], h, pad], axis=-1)

    return pl.pallas_call(
        body,
        grid=(grid,),
        in_specs=[
            pl.BlockSpec((nb * 3, _TW), lambda i: (i, 0)),
            pl.BlockSpec((nb, _TW), lambda i: (i, 0)),
            pl.BlockSpec(Wd.shape, lambda i: (0, 0)),
            pl.BlockSpec((1, c_out), lambda i: (0, 0)),
        ],
        out_specs=pl.BlockSpec((nb, _TW), lambda i: (i, 0)),
        out_shape=jax.ShapeDtypeStruct((np_rows, _TW), jnp.float32),
    )(rows, skip, Wd, bd.reshape(1, c_out))


def _head_tc(t_d0, Wh1, bh1, gamma, beta, Wh2, bh2, n_valid, nb):
    np_rows = t_d0.shape[0]
    grid = np_rows // nb

    def body1(d0_ref, w_ref, b_ref, h_ref, st_ref):
        i = pl.program_id(0)
        hb = _dot16(d0_ref[...][:, 16:48], w_ref[...]) + b_ref[...]
        h_ref[...] = hb
        ridx = lax.broadcasted_iota(jnp.int32, (nb, 1), 0) + i * nb
        hm = jnp.where(ridx < n_valid, hb, 0.0)
        st = jnp.concatenate([jnp.sum(hm, axis=0, keepdims=True),
                              jnp.sum(hm * hm, axis=0, keepdims=True)], axis=0)

        @pl.when(i == 0)
        def _():
            st_ref[...] = jnp.zeros_like(st_ref)

        st_ref[...] += st

    h, st = pl.pallas_call(
        body1,
        grid=(grid,),
        in_specs=[
            pl.BlockSpec((nb, _TW), lambda i: (i, 0)),
            pl.BlockSpec(Wh1.shape, lambda i: (0, 0)),
            pl.BlockSpec((1, 32), lambda i: (0, 0)),
        ],
        out_specs=[
            pl.BlockSpec((nb, 32), lambda i: (i, 0)),
            pl.BlockSpec((2, 32), lambda i: (0, 0)),
        ],
        out_shape=[
            jax.ShapeDtypeStruct((np_rows, 32), jnp.float32),
            jax.ShapeDtypeStruct((2, 32), jnp.float32),
        ],
    )(t_d0, Wh1, bh1.reshape(1, 32))

    def body2(h_ref, st_ref, g_ref, be_ref, w2_ref, b2_ref, out_ref):
        stv = st_ref[...]
        mu = stv[0:1, :] * (1.0 / n_valid)
        ex2 = stv[1:2, :] * (1.0 / n_valid)
        var = ex2 - mu * mu
        hn = g_ref[...] * (h_ref[...] - mu) / jnp.sqrt(var + 1e-5) + be_ref[...]
        hn = jnp.maximum(hn, 0.0)
        out_ref[...] = _dot16(hn, w2_ref[...]) + b2_ref[...]

    return pl.pallas_call(
        body2,
        grid=(grid,),
        in_specs=[
            pl.BlockSpec((nb, 32), lambda i: (i, 0)),
            pl.BlockSpec((2, 32), lambda i: (0, 0)),
            pl.BlockSpec((1, 32), lambda i: (0, 0)),
            pl.BlockSpec((1, 32), lambda i: (0, 0)),
            pl.BlockSpec(Wh2.shape, lambda i: (0, 0)),
            pl.BlockSpec((1, 13), lambda i: (0, 0)),
        ],
        out_specs=pl.BlockSpec((nb, 13), lambda i: (i, 0)),
        out_shape=jax.ShapeDtypeStruct((np_rows, 13), jnp.float32),
    )(h, st, gamma.reshape(1, 32), beta.reshape(1, 32), Wh2, bh2.reshape(1, 13))


def _padtab(coords, np_rows):
    return jnp.pad(coords, ((0, np_rows - coords.shape[0]), (0, _TW - coords.shape[1])))


def kernel(x, points1, points2, nbr0, nbr1, nbr2, sub0, sub1, up0, up1, obs, We0_0, be0_0, We1_0, be1_0, We1_1, be1_1, We2_0, be2_0, We2_1, be2_1, We2_2, be2_2, Wp0, bp0, Wp1, bp1, Wd1, bd1, Wr1, br1, Wd0, bd0, Wr0, br0, Wh1, bh1, gamma, beta, Wh2, bh2):
    n0, n1, n2 = x.shape[0], points1.shape[0], points2.shape[0]
    nb0, nb1, nb2 = 256, 256, 128
    n0p = _cdiv(n0, nb0) * nb0
    n1p = _cdiv(n1, nb1) * nb1
    n2p = _cdiv(n2, nb2) * nb2

    dst0 = _padtab(x[:, :3], n0p)
    dst1 = _padtab(points1, n1p)
    dst2 = _padtab(points2, n2p)

    g = _sc_gather(dst0, nbr0, n0p)
    t_f0 = _gib_tc(g, dst0, obs, We0_0, be0_0, 3, 0, nb0, 16)

    g = _sc_gather(t_f0, sub0, n1p)
    t_p1 = _gib_tc(g, dst1, obs, Wp0, bp0, 32, 16, nb1, 16)

    g = _sc_gather(t_p1, nbr1, n1p)
    t_f1a = _gib_tc(g, dst1, obs, We1_0, be1_0, 32, 16, nb1, 16)

    g = _sc_gather(t_f1a, nbr1, n1p)
    t_f1 = _gib_tc(g, dst1, obs, We1_1, be1_1, 64, 16, nb1, 16)

    g = _sc_gather(t_f1, sub1, n2p)
    t_p2 = _gib_tc(g, dst2, obs, Wp1, bp1, 64, 16, nb2, 16)

    g = _sc_gather(t_p2, nbr2, n2p)
    t_f2a = _gib_tc(g, dst2, obs, We2_0, be2_0, 64, 16, nb2, 16)

    g = _sc_gather(t_f2a, nbr2, n2p)
    t_f2b = _gib_tc(g, dst2, obs, We2_1, be2_1, 96, 16, nb2, 16)

    g = _sc_gather(t_f2b, nbr2, n2p)
    t_f2 = _gib_tc(g, dst2, obs, We2_2, be2_2, 96, 16, nb2, 16)

    g = _sc_gather(t_f2, up1, n1p)
    t_h1 = _dec_tc(g, t_f1, 96, 64, Wd1, bd1, nb1)
    g = _sc_gather(t_h1, nbr1, n1p)
    t_d1 = _gib_tc(g, dst1, obs, Wr1, br1, 64, 16, nb1, 16)

    g = _sc_gather(t_d1, up0, n0p)
    t_h0 = _dec_tc(g, t_f0, 64, 32, Wd0, bd0, nb0)
    g = _sc_gather(t_h0, nbr0, n0p)
    t_d0 = _gib_tc(g, dst0, obs, Wr0, br0, 32, 16, nb0, 16)

    out = _head_tc(t_d0, Wh1, bh1, gamma, beta, Wh2, bh2, n0, nb0)
    return out[:n0]
